# baseline (speedup 1.0000x reference)
import functools
import jax
import jax.numpy as jnp
from jax import lax
from jax.experimental import pallas as pl
from jax.experimental.pallas import tpu as pltpu
from jax.experimental.pallas import tpu_sc as plsc

_N = 10000
_E = 320000
_C = 128
_BN = 1000
_BE = 2000

_NW = 32
_EPW = _E // _NW
_GB = 80
_GCH = _EPW // _GB


def _sc_gather(zdt, zst, val, pos4, src, dst):
    mesh = plsc.VectorSubcoreMesh(core_axis_name="c", subcore_axis_name="s")
    out_type = [jax.ShapeDtypeStruct((_E, _C), jnp.float32),
                jax.ShapeDtypeStruct((_E, _C), jnp.float32)]
    slot = [pltpu.VMEM((_GB,), jnp.int32),
            pltpu.VMEM((_GB,), jnp.int32),
            pltpu.VMEM((_GB, 64), jnp.float32),
            pltpu.VMEM((_GB, 64), jnp.float32),
            pltpu.VMEM((_GB, _C), jnp.float32),
            pltpu.VMEM((_GB, _C), jnp.float32),
            pltpu.SemaphoreType.DMA,
            pltpu.SemaphoreType.DMA,
            pltpu.SemaphoreType.DMA]
    scratch = slot + slot + [pltpu.VMEM((_N * 4,), jnp.float32)]

    @functools.partial(pl.kernel, mesh=mesh, out_type=out_type,
                       scratch_types=scratch,
                       compiler_params=pltpu.CompilerParams(
                           needs_layout_passes=False,
                           use_tc_tiling_on_sc=False))
    def k(zdt_h, zst_h, val_h, pos4_h, src_h, dst_h,
          zrel_h, gv_h,
          ixs0, ixd0, bd0, bs0, bv0, bzr0, semi0, semg0, semw0,
          ixs1, ixd1, bd1, bs1, bv1, bzr1, semi1, semg1, semw1,
          postab):
        wid = lax.axis_index("s") * 2 + lax.axis_index("c")
        base = wid * _EPW
        S = [(ixs0, ixd0, bd0, bs0, bv0, bzr0, semi0, semg0, semw0),
             (ixs1, ixd1, bd1, bs1, bv1, bzr1, semi1, semg1, semw1)]

        pltpu.sync_copy(pos4_h, postab)
        zeros = jnp.zeros((16,), jnp.float32)

        def zrow(g, carry):
            bzr0[g, pl.ds(64, 16)] = zeros
            bzr1[g, pl.ds(64, 16)] = zeros
            return carry
        lax.fori_loop(0, _GB, zrow, 0)

        lanes = lax.iota(jnp.int32, 16)

        def fire_idx(b, c):
            ixs, ixd = S[b][0], S[b][1]
            semi = S[b][6]
            off = base + c * _GB
            pltpu.async_copy(src_h.at[pl.ds(off, _GB)], ixs, semi)
            pltpu.async_copy(dst_h.at[pl.ds(off, _GB)], ixd, semi)

        def drain_idx(b):
            ixs, ixd = S[b][0], S[b][1]
            semi = S[b][6]
            pltpu.make_async_copy(src_h.at[pl.ds(base, _GB)], ixs, semi).wait()
            pltpu.make_async_copy(dst_h.at[pl.ds(base, _GB)], ixd, semi).wait()

        def fire_gather_combo(b):
            ixs, ixd, bd, bs = S[b][:4]
            semg = S[b][7]
            pltpu.async_copy(zdt_h.at[ixd], bd, semg)
            pltpu.async_copy(zst_h.at[ixs], bs, semg)

        def fire_gather_val(b):
            ixs, bv = S[b][0], S[b][4]
            semg = S[b][7]
            pltpu.async_copy(val_h.at[ixs], bv, semg)

        def drain_gather(b):
            ixs, ixd, bd, bs, bv = S[b][:5]
            semg = S[b][7]
            pltpu.make_async_copy(zdt_h.at[ixd], bd, semg).wait()
            pltpu.make_async_copy(zst_h.at[ixs], bs, semg).wait()
            pltpu.make_async_copy(val_h.at[ixs], bv, semg).wait()

        def fire_write(b, c):
            bv, bzr = S[b][4], S[b][5]
            semw = S[b][8]
            off = base + c * _GB
            pltpu.async_copy(bzr, zrel_h.at[pl.ds(off, _GB)], semw)
            pltpu.async_copy(bv, gv_h.at[pl.ds(off, _GB)], semw)

        def drain_write(b):
            bv, bzr = S[b][4], S[b][5]
            semw = S[b][8]
            pltpu.make_async_copy(bzr, zrel_h.at[pl.ds(base, _GB)],
                                  semw).wait()
            pltpu.make_async_copy(bv, gv_h.at[pl.ds(base, _GB)], semw).wait()

        def compute(b):
            ixs, ixd, bd, bs, bv, bzr = S[b][:6]

            def group(g, carry2):
                id16 = ixd[pl.ds(g * 16, 16)] * 4
                is16 = ixs[pl.ds(g * 16, 16)] * 4
                rows = lanes + g * 16
                for col in range(3):
                    pd = plsc.load_gather(postab, [id16 + col])
                    ps = plsc.load_gather(postab, [is16 + col])
                    plsc.store_scatter(bzr, [rows, jnp.full((16,), 64 + col,
                                                            jnp.int32)],
                                       pd - ps)
                return carry2
            lax.fori_loop(0, _GB // 16, group, 0)

            def zsub(r, carry2):
                for j in range(4):
                    bzr[r, pl.ds(j * 16, 16)] = (
                        bd[r, pl.ds(j * 16, 16)]
                        - bs[r, pl.ds(j * 16, 16)])
                return carry2
            lax.fori_loop(0, _GB, zsub, 0)

        fire_idx(0, 0)
        fire_idx(1, 1)
        drain_idx(0)
        fire_gather_combo(0)
        fire_gather_val(0)

        def step(b, c, i2):
            bo = 1 - b
            drain_idx(bo)
            fire_gather_combo(bo)
            if b == 0:
                @pl.when(i2 > 0)
                def _():
                    drain_write(bo)
            else:
                drain_write(bo)
            fire_gather_val(bo)
            drain_gather(b)
            compute(b)
            fire_write(b, c)
            if b == 0:
                fire_idx(b, c + 2)
            else:
                @pl.when(c + 2 < _GCH)
                def _():
                    fire_idx(b, c + 2)

        def body(i2, carry):
            c0 = i2 * 2
            step(0, c0, i2)
            step(1, c0 + 1, i2)
            return carry
        lax.fori_loop(0, (_GCH - 1) // 2, body, 0)

        drain_gather(0)
        compute(0)
        fire_write(0, _GCH - 1)
        drain_write(1)
        drain_write(0)

    return k(zdt, zst, val, pos4, src, dst)


_SB = 80
_EPS = _E // 16
_SCH = _EPS // _SB
_NPAD = 10240
_NPT = _NPAD // 16


def _sc_scatter(e, m, dst, zeros):
    mesh = plsc.VectorSubcoreMesh(core_axis_name="c", subcore_axis_name="s")
    out_type = jax.ShapeDtypeStruct((2, _NPAD, _C), jnp.float32)
    scratch = [pltpu.VMEM_SHARED((_NPAD, _C), jnp.float32),
               pltpu.VMEM((_SB,), jnp.int32),
               pltpu.VMEM((_SB,), jnp.int32),
               pltpu.VMEM((_SB, _C), jnp.float32),
               pltpu.VMEM((_SB, _C), jnp.float32),
               pltpu.SemaphoreType.DMA,
               pltpu.SemaphoreType.DMA]

    @functools.partial(pl.kernel, mesh=mesh, out_type=out_type,
                       scratch_types=scratch)
    def k(e_h, m_h, dst_h, z_h, out_h, acc_sh, ix0, ix1, rw0, rw1,
          sem0, sem1):
        c = lax.axis_index("c")
        s = lax.axis_index("s")
        pltpu.sync_copy(z_h.at[pl.ds(s * _NPT, _NPT)],
                        acc_sh.at[pl.ds(s * _NPT, _NPT)])
        plsc.subcore_barrier()
        base = s * _EPS

        def sweep(rows_h):
            def fire(ix, rw, sem, ch):
                off = base + ch * _SB
                pltpu.async_copy(dst_h.at[pl.ds(off, _SB)], ix, sem)
                pltpu.async_copy(rows_h.at[pl.ds(off, _SB)], rw, sem)

            def drain(ix, rw, sem):
                pltpu.make_async_copy(dst_h.at[pl.ds(base, _SB)], ix,
                                      sem).wait()
                pltpu.make_async_copy(rows_h.at[pl.ds(base, _SB)], rw,
                                      sem).wait()

            fire(ix0, rw0, sem0, 0)

            def body(i2, carry):
                c0 = i2 * 2
                fire(ix1, rw1, sem1, c0 + 1)
                drain(ix0, rw0, sem0)
                pltpu.sync_copy(rw0, acc_sh.at[ix0], add=True)

                @pl.when(c0 + 2 < _SCH)
                def _():
                    fire(ix0, rw0, sem0, c0 + 2)
                drain(ix1, rw1, sem1)
                pltpu.sync_copy(rw1, acc_sh.at[ix1], add=True)
                return carry
            lax.fori_loop(0, _SCH // 2, body, 0)

        @pl.when(c == 0)
        def _():
            sweep(m_h)

        @pl.when(c == 1)
        def _():
            sweep(e_h)

        plsc.subcore_barrier()
        pltpu.sync_copy(acc_sh.at[pl.ds(s * _NPT, _NPT)],
                        out_h.at[c, pl.ds(s * _NPT, _NPT)])

    return k(e, m, dst, zeros)


def _node_body(x_ref, win_ref, bin_ref, wz_ref, wval_ref,
               zd_ref, zs_ref, val_ref):
    h = jnp.maximum(
        jnp.dot(x_ref[...], win_ref[...], preferred_element_type=jnp.float32)
        + bin_ref[...], 0.0)
    combo = jnp.dot(h, wz_ref[...], preferred_element_type=jnp.float32)
    zd_ref[...] = combo[:, :64]
    zs_ref[...] = combo[:, 64:]
    val_ref[...] = jnp.dot(h, wval_ref[...], preferred_element_type=jnp.float32)


def _node_call(x, W_in, b_in, Wz, W_val):
    full = lambda i: (0, 0)
    row = lambda i: (i, 0)
    spec_w = pl.BlockSpec((_C, _C), full)
    spec_b = pl.BlockSpec((1, _C), full)
    spec_n = pl.BlockSpec((_BN, _C), row)
    spec_h = pl.BlockSpec((_BN, 64), row)
    return pl.pallas_call(
        _node_body,
        grid=(_N // _BN,),
        in_specs=[spec_n, spec_w, spec_b, spec_w, spec_w],
        out_specs=[spec_h, spec_h, spec_n],
        out_shape=[jax.ShapeDtypeStruct((_N, 64), jnp.float32),
                   jax.ShapeDtypeStruct((_N, 64), jnp.float32),
                   jax.ShapeDtypeStruct((_N, _C), jnp.float32)],
    )(x, W_in, b_in, Wz, W_val)


def _edge_body(zrel_ref, gv_ref,
               wp1_ref, bp1_ref, wpa_ref, bpa_ref,
               wp2_ref, bp2_ref, wa2_ref, ba2_ref,
               e_ref, m_ref):
    z = zrel_ref[:, :64]
    rel = zrel_ref[:, 64:80]
    d1 = jnp.maximum(
        jnp.dot(rel, wp1_ref[...], preferred_element_type=jnp.float32)
        + bp1_ref[...], 0.0)
    delta = jnp.dot(d1, wp2_ref[...], preferred_element_type=jnp.float32) \
        + bp2_ref[...]
    u = jnp.maximum(
        z + jnp.dot(d1, wpa_ref[...], preferred_element_type=jnp.float32)
        + bpa_ref[...], 0.0)
    a = jnp.dot(u, wa2_ref[...], preferred_element_type=jnp.float32) \
        + ba2_ref[...]
    e = jnp.exp(a)
    e_ref[...] = e
    m_ref[...] = e * (gv_ref[...] + delta)


def _edge_call(zrel, gv, W_p1p, b_p1, W_pa, b_pa, W_p2, b_p2, W_a2, b_a2):
    full = lambda i: (0, 0)
    row = lambda i: (i, 0)
    spec_e = pl.BlockSpec((_BE, _C), row)
    return pl.pallas_call(
        _edge_body,
        grid=(_E // _BE,),
        in_specs=[spec_e, spec_e,
                  pl.BlockSpec((16, 64), full), pl.BlockSpec((1, 64), full),
                  pl.BlockSpec((64, 64), full), pl.BlockSpec((1, 64), full),
                  pl.BlockSpec((64, _C), full), pl.BlockSpec((1, _C), full),
                  pl.BlockSpec((64, _C), full), pl.BlockSpec((1, _C), full)],
        out_specs=[spec_e, spec_e],
        out_shape=[jax.ShapeDtypeStruct((_E, _C), jnp.float32)] * 2,
    )(zrel, gv, W_p1p, b_p1, W_pa, b_pa, W_p2, b_p2, W_a2, b_a2)


def _final_body(num_ref, den_ref, x_ref, wout_ref, bout_ref, out_ref):
    agg = num_ref[...] / (den_ref[...] + 1e-16)
    out_ref[...] = jnp.maximum(
        jnp.dot(agg, wout_ref[...], preferred_element_type=jnp.float32)
        + bout_ref[...], 0.0) + x_ref[...]


def _final_call(num, den, x, W_out, b_out):
    full = lambda i: (0, 0)
    row = lambda i: (i, 0)
    spec_n = pl.BlockSpec((_BN, _C), row)
    return pl.pallas_call(
        _final_body,
        grid=(_N // _BN,),
        in_specs=[spec_n, spec_n, spec_n,
                  pl.BlockSpec((_C, _C), full), pl.BlockSpec((1, _C), full)],
        out_specs=spec_n,
        out_shape=jax.ShapeDtypeStruct((_N, _C), jnp.float32),
    )(num, den, x, W_out, b_out)


def kernel(x, pos, edge_index, W_in, b_in, W_val, W_src, W_dst,
           W_p1, b_p1, W_p2, b_p2, W_a1, b_a1, W_a2, b_a2, W_out, b_out):
    b_in2 = b_in.reshape(1, _C)
    b_p12 = b_p1.reshape(1, 64)
    b_p22 = b_p2.reshape(1, _C)
    b_a22 = b_a2.reshape(1, _C)
    b_out2 = b_out.reshape(1, _C)
    W_p1p = jnp.pad(W_p1, ((0, 13), (0, 0)))
    Wz = jnp.concatenate([W_dst @ W_a1, W_src @ W_a1], axis=1)
    W_pa = W_p2 @ W_a1
    b_pa = (b_p2 @ W_a1 + b_a1).reshape(1, 64)

    zdt, zst, val = _node_call(x, W_in, b_in2, Wz, W_val)

    src = edge_index[0]
    dst = edge_index[1]
    pos4 = jnp.pad(pos, ((0, 0), (0, 1))).reshape(-1)
    zrel, gv = _sc_gather(zdt, zst, val, pos4, src, dst)

    e, m = _edge_call(zrel, gv, W_p1p, b_p12, W_pa, b_pa, W_p2, b_p22,
                      W_a2, b_a22)

    acc = _sc_scatter(e, m, dst, jnp.zeros((_NPAD, _C), jnp.float32))

    return _final_call(acc[0, :_N], acc[1, :_N], x, W_out, b_out2)


# speedup vs baseline: 1.1709x; 1.1709x over previous
import functools
import jax
import jax.numpy as jnp
from jax import lax
from jax.experimental import pallas as pl
from jax.experimental.pallas import tpu as pltpu
from jax.experimental.pallas import tpu_sc as plsc

_N = 10000
_E = 320000
_C = 128
_BN = 1000
_BE = 2000

_NW = 32
_EPW = _E // _NW
_GB = 80
_GCH = _EPW // _GB


def _sc_gather(combo, val, pos4, src, dst, ne, gch, tok):
    epw = ne // _NW
    mesh = plsc.VectorSubcoreMesh(core_axis_name="c", subcore_axis_name="s")
    out_type = [jax.ShapeDtypeStruct((ne, _C), jnp.float32),
                jax.ShapeDtypeStruct((ne, _C), jnp.float32)]
    slot = [pltpu.VMEM((_GB,), jnp.int32),
            pltpu.VMEM((_GB,), jnp.int32),
            pltpu.VMEM((_GB, _C), jnp.float32),
            pltpu.VMEM((_GB, _C), jnp.float32),
            pltpu.VMEM((_GB, _C), jnp.float32),
            pltpu.VMEM((_GB, _C), jnp.float32),
            pltpu.SemaphoreType.DMA,
            pltpu.SemaphoreType.DMA,
            pltpu.SemaphoreType.DMA]
    scratch = slot + slot + [pltpu.VMEM((_N * 4,), jnp.float32)]

    @functools.partial(pl.kernel, mesh=mesh, out_type=out_type,
                       scratch_types=scratch,
                       compiler_params=pltpu.CompilerParams(
                           needs_layout_passes=False))
    def k(combo_h, val_h, pos4_h, src_h, dst_h, tok_h,
          zrel_h, gv_h,
          ixs0, ixd0, bd0, bs0, bv0, bzr0, semi0, semg0, semw0,
          ixs1, ixd1, bd1, bs1, bv1, bzr1, semi1, semg1, semw1,
          postab):
        wid = lax.axis_index("s") * 2 + lax.axis_index("c")
        base = wid * epw
        S = [(ixs0, ixd0, bd0, bs0, bv0, bzr0, semi0, semg0, semw0),
             (ixs1, ixd1, bd1, bs1, bv1, bzr1, semi1, semg1, semw1)]

        pltpu.sync_copy(pos4_h, postab)
        zeros = jnp.zeros((16,), jnp.float32)

        def zrow(g, carry):
            bzr0[g, pl.ds(64, 16)] = zeros
            bzr1[g, pl.ds(64, 16)] = zeros
            return carry
        lax.fori_loop(0, _GB, zrow, 0)

        lanes = lax.iota(jnp.int32, 16)

        def fire_idx(b, c):
            ixs, ixd = S[b][0], S[b][1]
            semi = S[b][6]
            off = base + c * _GB
            pltpu.async_copy(src_h.at[pl.ds(off, _GB)], ixs, semi)
            pltpu.async_copy(dst_h.at[pl.ds(off, _GB)], ixd, semi)

        def drain_idx(b):
            ixs, ixd = S[b][0], S[b][1]
            semi = S[b][6]
            pltpu.make_async_copy(src_h.at[pl.ds(base, _GB)], ixs, semi).wait()
            pltpu.make_async_copy(dst_h.at[pl.ds(base, _GB)], ixd, semi).wait()

        def fire_gather_combo(b):
            ixs, ixd, bd, bs = S[b][:4]
            semg = S[b][7]
            pltpu.async_copy(combo_h.at[ixd], bd, semg)
            pltpu.async_copy(combo_h.at[ixs], bs, semg)

        def fire_gather_val(b):
            ixs, bv = S[b][0], S[b][4]
            semg = S[b][7]
            pltpu.async_copy(val_h.at[ixs], bv, semg)

        def drain_gather(b):
            ixs, ixd, bd, bs, bv = S[b][:5]
            semg = S[b][7]
            pltpu.make_async_copy(combo_h.at[ixd], bd, semg).wait()
            pltpu.make_async_copy(combo_h.at[ixs], bs, semg).wait()
            pltpu.make_async_copy(val_h.at[ixs], bv, semg).wait()

        def fire_write(b, c):
            bv, bzr = S[b][4], S[b][5]
            semw = S[b][8]
            off = base + c * _GB
            pltpu.async_copy(bzr, zrel_h.at[pl.ds(off, _GB)], semw)
            pltpu.async_copy(bv, gv_h.at[pl.ds(off, _GB)], semw)

        def drain_write(b):
            bv, bzr = S[b][4], S[b][5]
            semw = S[b][8]
            pltpu.make_async_copy(bzr, zrel_h.at[pl.ds(base, _GB)],
                                  semw).wait()
            pltpu.make_async_copy(bv, gv_h.at[pl.ds(base, _GB)], semw).wait()

        def compute(b):
            ixs, ixd, bd, bs, bv, bzr = S[b][:6]

            def group(g, carry2):
                id16 = ixd[pl.ds(g * 16, 16)] * 4
                is16 = ixs[pl.ds(g * 16, 16)] * 4
                rows = lanes + g * 16
                for col in range(3):
                    pd = plsc.load_gather(postab, [id16 + col])
                    ps = plsc.load_gather(postab, [is16 + col])
                    plsc.store_scatter(bzr, [rows, jnp.full((16,), 64 + col,
                                                            jnp.int32)],
                                       pd - ps)
                return carry2
            lax.fori_loop(0, _GB // 16, group, 0)

            def zsub(r, carry2):
                for j in range(4):
                    bzr[r, pl.ds(j * 16, 16)] = (
                        bd[r, pl.ds(j * 16, 16)]
                        - bs[r, pl.ds(64 + j * 16, 16)])
                return carry2
            lax.fori_loop(0, _GB, zsub, 0)

        fire_idx(0, 0)
        fire_idx(1, 1)
        drain_idx(0)
        fire_gather_combo(0)
        fire_gather_val(0)

        def step(b, c, i2):
            bo = 1 - b
            drain_idx(bo)
            fire_gather_combo(bo)
            if b == 0:
                @pl.when(i2 > 0)
                def _():
                    drain_write(bo)
            else:
                drain_write(bo)
            fire_gather_val(bo)
            drain_gather(b)
            compute(b)
            fire_write(b, c)
            if b == 0:
                fire_idx(b, c + 2)
            else:
                @pl.when(c + 2 < gch)
                def _():
                    fire_idx(b, c + 2)

        def body(i2, carry):
            c0 = i2 * 2
            step(0, c0, i2)
            step(1, c0 + 1, i2)
            return carry

        if gch % 2 == 1:
            lax.fori_loop(0, (gch - 1) // 2, body, 0)
            drain_gather(0)
            compute(0)
            fire_write(0, gch - 1)
            drain_write(1)
            drain_write(0)
        else:
            lax.fori_loop(0, gch // 2 - 1, body, 0)
            drain_idx(1)
            fire_gather_combo(1)
            drain_write(1)
            fire_gather_val(1)
            drain_gather(0)
            compute(0)
            fire_write(0, gch - 2)
            drain_gather(1)
            compute(1)
            fire_write(1, gch - 1)
            drain_write(0)
            drain_write(1)

    return k(combo, val, pos4, src, dst, tok)


_SLABS = ((0, 161280, 63, 1920), (161280, 158720, 62, 1984))

_SB = 80
_EPS = _E // 16
_SCH = _EPS // _SB
_NPAD = 10240
_NPT = _NPAD // 16


def _sc_scatter(e, m, dst, zeros, ne, tok):
    eps = ne // 16
    sch = eps // _SB
    mesh = plsc.VectorSubcoreMesh(core_axis_name="c", subcore_axis_name="s")
    out_type = jax.ShapeDtypeStruct((2, _NPAD, _C), jnp.float32)
    scratch = [pltpu.VMEM_SHARED((_NPAD, _C), jnp.float32),
               pltpu.VMEM((_SB,), jnp.int32),
               pltpu.VMEM((_SB,), jnp.int32),
               pltpu.VMEM((_SB, _C), jnp.float32),
               pltpu.VMEM((_SB, _C), jnp.float32),
               pltpu.SemaphoreType.DMA,
               pltpu.SemaphoreType.DMA]

    @functools.partial(pl.kernel, mesh=mesh, out_type=out_type,
                       scratch_types=scratch)
    def k(e_h, m_h, dst_h, z_h, tok_h, out_h, acc_sh, ix0, ix1, rw0, rw1,
          sem0, sem1):
        c = lax.axis_index("c")
        s = lax.axis_index("s")
        pltpu.sync_copy(z_h.at[pl.ds(s * _NPT, _NPT)],
                        acc_sh.at[pl.ds(s * _NPT, _NPT)])
        plsc.subcore_barrier()
        base = s * eps

        def sweep(rows_h):
            def fire(ix, rw, sem, ch):
                off = base + ch * _SB
                pltpu.async_copy(dst_h.at[pl.ds(off, _SB)], ix, sem)
                pltpu.async_copy(rows_h.at[pl.ds(off, _SB)], rw, sem)

            def drain(ix, rw, sem):
                pltpu.make_async_copy(dst_h.at[pl.ds(base, _SB)], ix,
                                      sem).wait()
                pltpu.make_async_copy(rows_h.at[pl.ds(base, _SB)], rw,
                                      sem).wait()

            fire(ix0, rw0, sem0, 0)

            def body(i2, carry):
                c0 = i2 * 2
                fire(ix1, rw1, sem1, c0 + 1)
                drain(ix0, rw0, sem0)
                pltpu.sync_copy(rw0, acc_sh.at[ix0], add=True)

                @pl.when(c0 + 2 < sch)
                def _():
                    fire(ix0, rw0, sem0, c0 + 2)
                drain(ix1, rw1, sem1)
                pltpu.sync_copy(rw1, acc_sh.at[ix1], add=True)
                return carry
            lax.fori_loop(0, sch // 2, body, 0)

        @pl.when(c == 0)
        def _():
            sweep(m_h)

        @pl.when(c == 1)
        def _():
            sweep(e_h)

        plsc.subcore_barrier()
        pltpu.sync_copy(acc_sh.at[pl.ds(s * _NPT, _NPT)],
                        out_h.at[c, pl.ds(s * _NPT, _NPT)])

    return k(e, m, dst, zeros, tok)


def _node_body(x_ref, win_ref, bin_ref, wz_ref, wval_ref,
               combo_ref, val_ref):
    h = jnp.maximum(
        jnp.dot(x_ref[...], win_ref[...], preferred_element_type=jnp.float32)
        + bin_ref[...], 0.0)
    combo_ref[...] = jnp.dot(h, wz_ref[...], preferred_element_type=jnp.float32)
    val_ref[...] = jnp.dot(h, wval_ref[...], preferred_element_type=jnp.float32)


def _node_call(x, W_in, b_in, Wz, W_val):
    full = lambda i: (0, 0)
    row = lambda i: (i, 0)
    spec_w = pl.BlockSpec((_C, _C), full)
    spec_b = pl.BlockSpec((1, _C), full)
    spec_n = pl.BlockSpec((_BN, _C), row)
    return pl.pallas_call(
        _node_body,
        grid=(_N // _BN,),
        in_specs=[spec_n, spec_w, spec_b, spec_w, spec_w],
        out_specs=[spec_n, spec_n],
        out_shape=[jax.ShapeDtypeStruct((_N, _C), jnp.float32)] * 2,
    )(x, W_in, b_in, Wz, W_val)


def _edge_body(zrel_ref, gv_ref,
               wp1_ref, bp1_ref, wpa_ref, bpa_ref,
               wp2_ref, bp2_ref, wa2_ref, ba2_ref,
               e_ref, m_ref):
    z = zrel_ref[:, :64]
    rel = zrel_ref[:, 64:80]
    d1 = jnp.maximum(
        jnp.dot(rel, wp1_ref[...], preferred_element_type=jnp.float32)
        + bp1_ref[...], 0.0)
    delta = jnp.dot(d1, wp2_ref[...], preferred_element_type=jnp.float32) \
        + bp2_ref[...]
    u = jnp.maximum(
        z + jnp.dot(d1, wpa_ref[...], preferred_element_type=jnp.float32)
        + bpa_ref[...], 0.0)
    a = jnp.dot(u, wa2_ref[...], preferred_element_type=jnp.float32) \
        + ba2_ref[...]
    e = jnp.exp(a)
    e_ref[...] = e
    m_ref[...] = e * (gv_ref[...] + delta)


def _edge_call(zrel, gv, W_p1p, b_p1, W_pa, b_pa, W_p2, b_p2, W_a2, b_a2,
               ne, be):
    full = lambda i: (0, 0)
    row = lambda i: (i, 0)
    spec_e = pl.BlockSpec((be, _C), row)
    return pl.pallas_call(
        _edge_body,
        grid=(ne // be,),
        in_specs=[spec_e, spec_e,
                  pl.BlockSpec((16, 64), full), pl.BlockSpec((1, 64), full),
                  pl.BlockSpec((64, 64), full), pl.BlockSpec((1, 64), full),
                  pl.BlockSpec((64, _C), full), pl.BlockSpec((1, _C), full),
                  pl.BlockSpec((64, _C), full), pl.BlockSpec((1, _C), full)],
        out_specs=[spec_e, spec_e],
        out_shape=[jax.ShapeDtypeStruct((ne, _C), jnp.float32)] * 2,
    )(zrel, gv, W_p1p, b_p1, W_pa, b_pa, W_p2, b_p2, W_a2, b_a2)


def _final_body(n0_ref, d0_ref, n1_ref, d1_ref, x_ref, wout_ref, bout_ref,
                out_ref):
    agg = (n0_ref[...] + n1_ref[...]) / (d0_ref[...] + d1_ref[...] + 1e-16)
    out_ref[...] = jnp.maximum(
        jnp.dot(agg, wout_ref[...], preferred_element_type=jnp.float32)
        + bout_ref[...], 0.0) + x_ref[...]


def _final_call(n0, d0, n1, d1, x, W_out, b_out):
    full = lambda i: (0, 0)
    row = lambda i: (i, 0)
    spec_n = pl.BlockSpec((_BN, _C), row)
    return pl.pallas_call(
        _final_body,
        grid=(_N // _BN,),
        in_specs=[spec_n, spec_n, spec_n, spec_n, spec_n,
                  pl.BlockSpec((_C, _C), full), pl.BlockSpec((1, _C), full)],
        out_specs=spec_n,
        out_shape=jax.ShapeDtypeStruct((_N, _C), jnp.float32),
    )(n0, d0, n1, d1, x, W_out, b_out)


def kernel(x, pos, edge_index, W_in, b_in, W_val, W_src, W_dst,
           W_p1, b_p1, W_p2, b_p2, W_a1, b_a1, W_a2, b_a2, W_out, b_out):
    b_in2 = b_in.reshape(1, _C)
    b_p12 = b_p1.reshape(1, 64)
    b_p22 = b_p2.reshape(1, _C)
    b_a22 = b_a2.reshape(1, _C)
    b_out2 = b_out.reshape(1, _C)
    W_p1p = jnp.pad(W_p1, ((0, 13), (0, 0)))
    Wz = jnp.concatenate([W_dst @ W_a1, W_src @ W_a1], axis=1)
    W_pa = W_p2 @ W_a1
    b_pa = (b_p2 @ W_a1 + b_a1).reshape(1, 64)

    combo, val = _node_call(x, W_in, b_in2, Wz, W_val)

    src = edge_index[0]
    dst = edge_index[1]
    pos4 = jnp.pad(pos, ((0, 0), (0, 1))).reshape(-1)
    zeros = jnp.zeros((_NPAD, _C), jnp.float32)

    accs = []
    tok = jnp.zeros((8, _C), jnp.float32)
    for (lo, ne, gch, be) in _SLABS:
        src_s = lax.slice_in_dim(src, lo, lo + ne)
        dst_s = lax.slice_in_dim(dst, lo, lo + ne)
        zrel, gv = _sc_gather(combo, val, pos4, src_s, dst_s, ne, gch, tok)
        tok = lax.slice_in_dim(gv, 0, 8)
        e, m = _edge_call(zrel, gv, W_p1p, b_p12, W_pa, b_pa, W_p2, b_p22,
                          W_a2, b_a22, ne, be)
        accs.append((e, m, dst_s, ne))
    res = []
    for (e, m, dst_s, ne) in accs:
        acc = _sc_scatter(e, m, dst_s, zeros, ne, tok)
        tok = lax.slice_in_dim(acc[0], 0, 8)
        res.append(acc)
    acc0, acc1 = res

    return _final_call(acc0[0, :_N], acc0[1, :_N], acc1[0, :_N],
                       acc1[1, :_N], x, W_out, b_out2)
